# baseline (device time: 857869 ns/iter reference)
import jax
import jax.numpy as jnp
from jax import lax
from jax.experimental import pallas as pl
from jax.experimental.pallas import tpu as pltpu

N_DEV = 32
T = 256
D_SH = 4096
N_SUB = 2
D_SUB = D_SH // N_SUB
N_HOP = N_DEV - 1
LOG2 = 5


def kernel(x, W):
    logits = jnp.dot(
        x.astype(jnp.bfloat16),
        W.astype(jnp.bfloat16),
        preferred_element_type=jnp.float32,
    ).astype(jnp.bfloat16)

    def body(l_ref, out_ref, copy_sem, send_sems, recv_sems,
             stats_send, stats_recv, ssend_sems, srecv_sems,
             tin_ref, tout_ref, tsem):
        my = lax.axis_index("i")
        left = lax.rem(my + N_DEV - 1, N_DEV)
        right = lax.rem(my + 1, N_DEV)

        partners = [my ^ (1 << k) for k in range(1, LOG2)]
        barrier = pltpu.get_barrier_semaphore()
        for nbr in [left, right] + partners:
            pl.semaphore_signal(
                barrier, inc=1,
                device_id=(nbr,), device_id_type=pl.DeviceIdType.MESH,
            )
        pl.semaphore_wait(barrier, 2 + len(partners))

        def sub_ref(o, j):
            return out_ref.at[:, pl.ds(o * D_SH + j * D_SUB, D_SUB)]

        def origin(h):
            return lax.rem(my + (N_DEV - h), N_DEV)

        def mk(h, j):
            o = origin(h)
            return pltpu.make_async_remote_copy(
                src_ref=sub_ref(o, j),
                dst_ref=sub_ref(o, j),
                send_sem=send_sems.at[h * N_SUB + j],
                recv_sem=recv_sems.at[h * N_SUB + j],
                device_id=(right,),
                device_id_type=pl.DeviceIdType.MESH,
            )

        cp = pltpu.make_async_copy(
            l_ref, out_ref.at[:, pl.ds(my * D_SH, D_SH)], copy_sem)
        cp.start()
        cp.wait()
        descs = {}
        for j in range(N_SUB):
            descs[(0, j)] = mk(0, j)
            descs[(0, j)].start()

        s = jnp.sum(jnp.exp(l_ref[:, :].astype(jnp.float32)),
                    axis=1, keepdims=True)
        for k in range(LOG2):
            stats_send[k, :, :] = s
            p = my ^ (1 << k)
            d = pltpu.make_async_remote_copy(
                src_ref=stats_send.at[k],
                dst_ref=stats_recv.at[k],
                send_sem=ssend_sems.at[k],
                recv_sem=srecv_sems.at[k],
                device_id=(p,),
                device_id_type=pl.DeviceIdType.MESH,
            )
            d.start()
            d.wait()
            s = s + stats_recv[k, :, :]
        inv = 1.0 / s

        def transform(o):
            c = out_ref.at[:, pl.ds(o * D_SH, D_SH)]
            ci = pltpu.make_async_copy(c, tin_ref, tsem)
            ci.start()
            ci.wait()
            tout_ref[:, :] = jnp.exp(tin_ref[:, :]) * inv.astype(jnp.bfloat16)
            co = pltpu.make_async_copy(tout_ref, c, tsem)
            co.start()
            co.wait()

        for h in range(1, N_HOP):
            for j in range(N_SUB):
                descs[(h - 1, j)].wait_recv()
            for j in range(N_SUB):
                descs[(h, j)] = mk(h, j)
                descs[(h, j)].start()
            if h >= 2:
                for j in range(N_SUB):
                    descs[(h - 2, j)].wait_send()
                transform(origin(h - 2))

        for j in range(N_SUB):
            descs[(N_HOP - 1, j)].wait_recv()
        for h in (N_HOP - 2, N_HOP - 1):
            for j in range(N_SUB):
                descs[(h, j)].wait_send()
            transform(origin(h))
        transform(origin(N_DEV - 1))

    return pl.pallas_call(
        body,
        out_shape=jax.ShapeDtypeStruct((T, N_DEV * D_SH), jnp.bfloat16),
        in_specs=[pl.BlockSpec(memory_space=pltpu.VMEM)],
        out_specs=pl.BlockSpec(memory_space=pl.ANY),
        scratch_shapes=[
            pltpu.SemaphoreType.DMA,
            pltpu.SemaphoreType.DMA((N_HOP * N_SUB,)),
            pltpu.SemaphoreType.DMA((N_HOP * N_SUB,)),
            pltpu.VMEM((LOG2, T, 1), jnp.float32),
            pltpu.VMEM((LOG2, T, 1), jnp.float32),
            pltpu.SemaphoreType.DMA((LOG2,)),
            pltpu.SemaphoreType.DMA((LOG2,)),
            pltpu.VMEM((T, D_SH), jnp.bfloat16),
            pltpu.VMEM((T, D_SH), jnp.bfloat16),
            pltpu.SemaphoreType.DMA,
        ],
        compiler_params=pltpu.CompilerParams(collective_id=0),
    )(logits)


# device time: 782502 ns/iter; 1.0963x vs baseline; 1.0963x over previous
import jax
import jax.numpy as jnp
from jax import lax
from jax.experimental import pallas as pl
from jax.experimental.pallas import tpu as pltpu

N_DEV = 32
T = 256
D_SH = 4096
N_SUB = 2
D_SUB = D_SH // N_SUB
N_HOP = N_DEV - 1
LOG2 = 5


def kernel(x, W):
    logits = jnp.dot(
        x.astype(jnp.bfloat16),
        W.astype(jnp.bfloat16),
        preferred_element_type=jnp.float32,
    ).astype(jnp.bfloat16)

    def body(l_ref, out_ref, raw_ref, copy_sem, send_sems, recv_sems,
             stats_send, stats_recv, ssend_sems, srecv_sems,
             tin_ref, tout_ref, tsem):
        my = lax.axis_index("i")
        left = lax.rem(my + N_DEV - 1, N_DEV)
        right = lax.rem(my + 1, N_DEV)

        partners = [my ^ (1 << k) for k in range(1, LOG2)]
        barrier = pltpu.get_barrier_semaphore()
        for nbr in [left, right] + partners:
            pl.semaphore_signal(
                barrier, inc=1,
                device_id=(nbr,), device_id_type=pl.DeviceIdType.MESH,
            )
        pl.semaphore_wait(barrier, 2 + len(partners))

        def sub_ref(o, j):
            return raw_ref.at[:, pl.ds(o * D_SH + j * D_SUB, D_SUB)]

        def origin(h):
            return lax.rem(my + (N_DEV - h), N_DEV)

        def mk(h, j):
            o = origin(h)
            return pltpu.make_async_remote_copy(
                src_ref=sub_ref(o, j),
                dst_ref=sub_ref(o, j),
                send_sem=send_sems.at[h * N_SUB + j],
                recv_sem=recv_sems.at[h * N_SUB + j],
                device_id=(right,),
                device_id_type=pl.DeviceIdType.MESH,
            )

        cp = pltpu.make_async_copy(
            l_ref, raw_ref.at[:, pl.ds(my * D_SH, D_SH)], copy_sem)
        cp.start()
        cp.wait()
        descs = {}
        for j in range(N_SUB):
            descs[(0, j)] = mk(0, j)
            descs[(0, j)].start()

        s = jnp.sum(jnp.exp(l_ref[:, :].astype(jnp.float32)),
                    axis=1, keepdims=True)
        for k in range(LOG2):
            stats_send[k, :, :] = s
            p = my ^ (1 << k)
            d = pltpu.make_async_remote_copy(
                src_ref=stats_send.at[k],
                dst_ref=stats_recv.at[k],
                send_sem=ssend_sems.at[k],
                recv_sem=srecv_sems.at[k],
                device_id=(p,),
                device_id_type=pl.DeviceIdType.MESH,
            )
            d.start()
            d.wait()
            s = s + stats_recv[k, :, :]
        inv_bf = (1.0 / s).astype(jnp.bfloat16)

        def transform(o):
            ci = pltpu.make_async_copy(
                raw_ref.at[:, pl.ds(o * D_SH, D_SH)], tin_ref, tsem)
            ci.start()
            ci.wait()
            tout_ref[:, :] = jnp.exp(tin_ref[:, :]) * inv_bf
            co = pltpu.make_async_copy(
                tout_ref, out_ref.at[:, pl.ds(o * D_SH, D_SH)], tsem)
            co.start()
            co.wait()

        for h in range(1, N_HOP):
            for j in range(N_SUB):
                descs[(h - 1, j)].wait_recv()
                descs[(h, j)] = mk(h, j)
                descs[(h, j)].start()
            transform(origin(h - 1))
        transform(origin(N_HOP - 1))

        for j in range(N_SUB):
            descs[(N_HOP - 1, j)].wait_recv()
        transform(origin(N_DEV - 1))

        for h in range(N_HOP):
            for j in range(N_SUB):
                descs[(h, j)].wait_send()

    out, _raw = pl.pallas_call(
        body,
        out_shape=[
            jax.ShapeDtypeStruct((T, N_DEV * D_SH), jnp.bfloat16),
            jax.ShapeDtypeStruct((T, N_DEV * D_SH), jnp.bfloat16),
        ],
        in_specs=[pl.BlockSpec(memory_space=pltpu.VMEM)],
        out_specs=[
            pl.BlockSpec(memory_space=pl.ANY),
            pl.BlockSpec(memory_space=pl.ANY),
        ],
        scratch_shapes=[
            pltpu.SemaphoreType.DMA,
            pltpu.SemaphoreType.DMA((N_HOP * N_SUB,)),
            pltpu.SemaphoreType.DMA((N_HOP * N_SUB,)),
            pltpu.VMEM((LOG2, T, 1), jnp.float32),
            pltpu.VMEM((LOG2, T, 1), jnp.float32),
            pltpu.SemaphoreType.DMA((LOG2,)),
            pltpu.SemaphoreType.DMA((LOG2,)),
            pltpu.VMEM((T, D_SH), jnp.bfloat16),
            pltpu.VMEM((T, D_SH), jnp.bfloat16),
            pltpu.SemaphoreType.DMA,
        ],
        compiler_params=pltpu.CompilerParams(collective_id=0),
    )(logits)
    return out


# device time: 781455 ns/iter; 1.0978x vs baseline; 1.0013x over previous
import jax
import jax.numpy as jnp
from jax import lax
from jax.experimental import pallas as pl
from jax.experimental.pallas import tpu as pltpu

N_DEV = 32
T = 256
D_SH = 4096
N_SUB = 2
T_SUB = T // N_SUB
N_HOP = N_DEV - 1
LOG2 = 5


def kernel(x, W):
    logits = jnp.dot(
        x.astype(jnp.bfloat16),
        W.astype(jnp.bfloat16),
        preferred_element_type=jnp.float32,
    ).astype(jnp.bfloat16)

    def body(l_ref, out_ref, raw_ref, copy_sem, send_sems, recv_sems,
             stats_send, stats_recv, ssend_sems, srecv_sems,
             tin_ref, tout_ref, tsem):
        my = lax.axis_index("i")
        left = lax.rem(my + N_DEV - 1, N_DEV)
        right = lax.rem(my + 1, N_DEV)

        partners = [my ^ (1 << k) for k in range(1, LOG2)]
        barrier = pltpu.get_barrier_semaphore()
        for nbr in [left, right] + partners:
            pl.semaphore_signal(
                barrier, inc=1,
                device_id=(nbr,), device_id_type=pl.DeviceIdType.MESH,
            )
        pl.semaphore_wait(barrier, 2 + len(partners))

        def sub_ref(o, j):
            return raw_ref.at[o, pl.ds(j * T_SUB, T_SUB), :]

        def origin(h):
            return lax.rem(my + (N_DEV - h), N_DEV)

        def mk(h, j):
            o = origin(h)
            return pltpu.make_async_remote_copy(
                src_ref=sub_ref(o, j),
                dst_ref=sub_ref(o, j),
                send_sem=send_sems.at[h * N_SUB + j],
                recv_sem=recv_sems.at[h * N_SUB + j],
                device_id=(right,),
                device_id_type=pl.DeviceIdType.MESH,
            )

        cp = pltpu.make_async_copy(l_ref, raw_ref.at[my], copy_sem)
        cp.start()
        cp.wait()
        descs = {}
        for j in range(N_SUB):
            descs[(0, j)] = mk(0, j)
            descs[(0, j)].start()

        s = jnp.sum(jnp.exp(l_ref[:, :].astype(jnp.float32)),
                    axis=1, keepdims=True)
        for k in range(LOG2):
            stats_send[k, :, :] = s
            p = my ^ (1 << k)
            d = pltpu.make_async_remote_copy(
                src_ref=stats_send.at[k],
                dst_ref=stats_recv.at[k],
                send_sem=ssend_sems.at[k],
                recv_sem=srecv_sems.at[k],
                device_id=(p,),
                device_id_type=pl.DeviceIdType.MESH,
            )
            d.start()
            d.wait()
            s = s + stats_recv[k, :, :]
        inv_bf = (1.0 / s).astype(jnp.bfloat16)

        def transform(o):
            ci = pltpu.make_async_copy(raw_ref.at[o], tin_ref, tsem)
            ci.start()
            ci.wait()
            tout_ref[:, :] = jnp.exp(tin_ref[:, :]) * inv_bf
            co = pltpu.make_async_copy(
                tout_ref, out_ref.at[:, pl.ds(o * D_SH, D_SH)], tsem)
            co.start()
            co.wait()

        for h in range(1, N_HOP):
            for j in range(N_SUB):
                descs[(h - 1, j)].wait_recv()
                descs[(h, j)] = mk(h, j)
                descs[(h, j)].start()
            transform(origin(h - 1))
        transform(origin(N_HOP - 1))

        for j in range(N_SUB):
            descs[(N_HOP - 1, j)].wait_recv()
        transform(origin(N_DEV - 1))

        for h in range(N_HOP):
            for j in range(N_SUB):
                descs[(h, j)].wait_send()

    out, _raw = pl.pallas_call(
        body,
        out_shape=[
            jax.ShapeDtypeStruct((T, N_DEV * D_SH), jnp.bfloat16),
            jax.ShapeDtypeStruct((N_DEV, T, D_SH), jnp.bfloat16),
        ],
        in_specs=[pl.BlockSpec(memory_space=pltpu.VMEM)],
        out_specs=[
            pl.BlockSpec(memory_space=pl.ANY),
            pl.BlockSpec(memory_space=pl.ANY),
        ],
        scratch_shapes=[
            pltpu.SemaphoreType.DMA,
            pltpu.SemaphoreType.DMA((N_HOP * N_SUB,)),
            pltpu.SemaphoreType.DMA((N_HOP * N_SUB,)),
            pltpu.VMEM((LOG2, T, 1), jnp.float32),
            pltpu.VMEM((LOG2, T, 1), jnp.float32),
            pltpu.SemaphoreType.DMA((LOG2,)),
            pltpu.SemaphoreType.DMA((LOG2,)),
            pltpu.VMEM((T, D_SH), jnp.bfloat16),
            pltpu.VMEM((T, D_SH), jnp.bfloat16),
            pltpu.SemaphoreType.DMA,
        ],
        compiler_params=pltpu.CompilerParams(collective_id=0),
    )(logits)
    return out
